# baseline (speedup 1.0000x reference)
import functools

import jax
import jax.numpy as jnp
from jax import lax
from jax.experimental import pallas as pl
from jax.experimental.pallas import tpu as pltpu
from jax.experimental.pallas import tpu_sc as plsc

_NC = 2
_NS = 16
_NW = _NC * _NS


def _tc_vocab_scalar(table, W1, b1_2d, w2a, b2_2d, vocab_pad):
    vocab, emb = table.shape
    blk = 16384
    grid = pl.cdiv(vocab, blk)

    def body(x_ref, w1_ref, b1_ref, w2_ref, b2_ref, o_ref):
        x = x_ref[...]
        z = jnp.dot(x, w1_ref[...], preferred_element_type=jnp.float32)
        z = jnp.maximum(z + b1_ref[...], 0.0)
        vcol = (
            jnp.dot(z, w2_ref[...], preferred_element_type=jnp.float32)
            + b2_ref[...]
        )
        o_ref[...] = vcol.reshape(blk // 128, 128)

    return pl.pallas_call(
        body,
        grid=(grid,),
        in_specs=[
            pl.BlockSpec((blk, emb), lambda i: (i, 0)),
            pl.BlockSpec((emb, 8), lambda i: (0, 0)),
            pl.BlockSpec((1, 8), lambda i: (0, 0)),
            pl.BlockSpec((8, 1), lambda i: (0, 0)),
            pl.BlockSpec((1, 1), lambda i: (0, 0)),
        ],
        out_specs=pl.BlockSpec((blk // 128, 128), lambda i: (i, 0)),
        out_shape=jax.ShapeDtypeStruct((vocab_pad // 128, 128), jnp.float32),
    )(table, W1, b1_2d, w2a, b2_2d)


def _tc_axpy_epilogue(g_pad, s, c_2d):
    B, L = s.shape
    Lp = g_pad.shape[1]
    rb = 512
    grid = B // rb
    assert B % rb == 0

    def body(g_ref, s_ref, c_ref, o_ref):
        o_ref[...] = g_ref[:, :L] + s_ref[...] * c_ref[...]

    return pl.pallas_call(
        body,
        grid=(grid,),
        in_specs=[
            pl.BlockSpec((rb, Lp), lambda i: (i, 0)),
            pl.BlockSpec((rb, L), lambda i: (i, 0)),
            pl.BlockSpec((1, 1), lambda i: (0, 0)),
        ],
        out_specs=pl.BlockSpec((rb, L), lambda i: (i, 0)),
        out_shape=jax.ShapeDtypeStruct((B, L), jnp.float32),
    )(g_pad, s, c_2d)


def _sc_gather(v1d, ids2d):
    rows = ids2d.shape[0]
    rows_per_w = rows // _NW
    nb = 16
    nchunk = rows_per_w // nb
    depth = 4
    pf = 2
    ntok = nb * 128
    assert rows_per_w % nb == 0 and nchunk % depth == 0

    mesh = plsc.VectorSubcoreMesh(core_axis_name="c", subcore_axis_name="s")
    ring = lambda ty: [ty] * depth

    @functools.partial(
        pl.kernel,
        out_type=jax.ShapeDtypeStruct((rows, 128), jnp.float32),
        mesh=mesh,
        scratch_types=[
            *ring(pltpu.VMEM((nb, 128), jnp.int32)),
            *ring(pltpu.VMEM((nb, 128), jnp.float32)),
            *ring(pltpu.SemaphoreType.DMA),
            *ring(pltpu.SemaphoreType.DMA),
            *ring(pltpu.SemaphoreType.DMA),
        ],
    )
    def sc_k(v_hbm, ids_hbm, out_hbm, *bufs):
        idx_b = bufs[0:depth]
        val_b = bufs[depth : 2 * depth]
        isem = bufs[2 * depth : 3 * depth]
        gsem = bufs[3 * depth : 4 * depth]
        osem = bufs[4 * depth : 5 * depth]

        wid = lax.axis_index("s") * _NC + lax.axis_index("c")
        r_base = wid * rows_per_w

        def drain(ref, sem, src):
            pltpu.make_async_copy(src, ref, sem).wait()

        def drain_idx(u):
            drain(idx_b[u], isem[u], ids_hbm.at[pl.ds(0, nb)])

        def fire_idx(c, u):
            pltpu.async_copy(
                ids_hbm.at[pl.ds(r_base + c * nb, nb)], idx_b[u], isem[u]
            )

        def fire(c, u):
            drain_idx(u)
            for j in range(nb):
                pltpu.async_copy(
                    v_hbm.at[idx_b[u].at[j]],
                    val_b[u].at[j],
                    gsem[u],
                )

        def process(c, u, head=False):
            drain(val_b[u], gsem[u], out_hbm.at[pl.ds(0, nb)])
            pltpu.async_copy(
                val_b[u], out_hbm.at[pl.ds(r_base + c * nb, nb)], osem[u]
            )
            fire_idx(jnp.minimum(c + depth, nchunk - 1), u)
            cp = c + pf
            up = (u + pf) % depth
            if not head:
                drain(val_b[up], osem[up], out_hbm.at[pl.ds(0, nb)])
            fire(jnp.minimum(cp, nchunk - 1), up)

        for u in range(depth):
            fire_idx(u, u)
        for u in range(pf):
            fire(u, u)
        for u in range(depth):
            process(u, u, head=u < pf)

        def group(g, carry):
            for u in range(depth):
                process(g * depth + u, u)
            return carry

        lax.fori_loop(1, nchunk // depth, group, 0)
        for u in (0, 1):
            drain(val_b[u], gsem[u], out_hbm.at[pl.ds(0, nb)])
        for u in (2, 3):
            drain_idx(u)
            drain(val_b[u], osem[u], out_hbm.at[pl.ds(0, nb)])

    return sc_k(v1d, ids2d)


def kernel(input_ids, s, emb_table, W1, b1, W2, b2):
    B, _, L = input_ids.shape
    vocab = emb_table.shape[0]
    bl = B * L
    assert bl % (128 * _NW) == 0
    vocab_pad = ((vocab + 16383) // 16384) * 16384

    v2d = _tc_vocab_scalar(
        emb_table, W1, b1.reshape(1, 8), W2[:8], b2.reshape(1, 1), vocab_pad
    )
    v1d = v2d.reshape(vocab_pad)
    Lp = ((L + 127) // 128) * 128
    ids_pad = jnp.pad(input_ids.reshape(B, L), ((0, 0), (0, Lp - L)))
    ids2d = ids_pad.reshape(B * Lp // 128, 128)

    g2d = _sc_gather(v1d, ids2d)
    g_pad = g2d.reshape(B, Lp)
    c_2d = W2[8].reshape(1, 1)
    return _tc_axpy_epilogue(g_pad, s, c_2d)


# speedup vs baseline: 9.9611x; 9.9611x over previous
import functools

import jax
import jax.numpy as jnp
from jax import lax
from jax.experimental import pallas as pl
from jax.experimental.pallas import tpu as pltpu
from jax.experimental.pallas import tpu_sc as plsc

_NC = 2
_NS = 16
_NW = _NC * _NS


def _tc_vocab_scalar(table, W1, b1_2d, w2a, b2_2d, vocab_pad):
    vocab, emb = table.shape
    blk = 16384
    grid = pl.cdiv(vocab, blk)

    def body(x_ref, w1_ref, b1_ref, w2_ref, b2_ref, o_ref):
        x = x_ref[...]
        z = jnp.dot(x, w1_ref[...], preferred_element_type=jnp.float32)
        z = jnp.maximum(z + b1_ref[...], 0.0)
        vcol = (
            jnp.dot(z, w2_ref[...], preferred_element_type=jnp.float32)
            + b2_ref[...]
        )
        o_ref[...] = vcol.reshape(blk // 128, 128)

    return pl.pallas_call(
        body,
        grid=(grid,),
        in_specs=[
            pl.BlockSpec((blk, emb), lambda i: (i, 0)),
            pl.BlockSpec((emb, 8), lambda i: (0, 0)),
            pl.BlockSpec((1, 8), lambda i: (0, 0)),
            pl.BlockSpec((8, 1), lambda i: (0, 0)),
            pl.BlockSpec((1, 1), lambda i: (0, 0)),
        ],
        out_specs=pl.BlockSpec((blk // 128, 128), lambda i: (i, 0)),
        out_shape=jax.ShapeDtypeStruct((vocab_pad // 128, 128), jnp.float32),
    )(table, W1, b1_2d, w2a, b2_2d)


def _tc_axpy_epilogue(g_pad, s, c_2d):
    B, L = s.shape
    Lp = g_pad.shape[1]
    rb = 512
    grid = B // rb
    assert B % rb == 0

    def body(g_ref, s_ref, c_ref, o_ref):
        o_ref[...] = g_ref[:, :L] + s_ref[...] * c_ref[...]

    return pl.pallas_call(
        body,
        grid=(grid,),
        in_specs=[
            pl.BlockSpec((rb, Lp), lambda i: (i, 0)),
            pl.BlockSpec((rb, L), lambda i: (i, 0)),
            pl.BlockSpec((1, 1), lambda i: (0, 0)),
        ],
        out_specs=pl.BlockSpec((rb, L), lambda i: (i, 0)),
        out_shape=jax.ShapeDtypeStruct((B, L), jnp.float32),
    )(g_pad, s, c_2d)


def _sc_gather(v1d, ids2d):
    rows = ids2d.shape[0]
    rows_per_w = rows // _NW
    nb = 16
    nchunk = rows_per_w // nb
    depth = 4
    pf = 2
    ntok = nb * 128
    assert rows_per_w % nb == 0 and nchunk % depth == 0

    mesh = plsc.VectorSubcoreMesh(core_axis_name="c", subcore_axis_name="s")
    ring = lambda ty: [ty] * depth

    @functools.partial(
        pl.kernel,
        out_type=jax.ShapeDtypeStruct((rows, 128), jnp.float32),
        mesh=mesh,
        scratch_types=[
            *ring(pltpu.VMEM((nb, 128), jnp.int32)),
            *ring(pltpu.VMEM((nb, 128), jnp.float32)),
            *ring(pltpu.SemaphoreType.DMA),
            *ring(pltpu.SemaphoreType.DMA),
            *ring(pltpu.SemaphoreType.DMA),
        ],
    )
    def sc_k(v_hbm, ids_hbm, out_hbm, *bufs):
        idx_b = bufs[0:depth]
        val_b = bufs[depth : 2 * depth]
        isem = bufs[2 * depth : 3 * depth]
        gsem = bufs[3 * depth : 4 * depth]
        osem = bufs[4 * depth : 5 * depth]

        wid = lax.axis_index("s") * _NC + lax.axis_index("c")
        r_base = wid * rows_per_w

        def drain(ref, sem, src):
            pltpu.make_async_copy(src, ref, sem).wait()

        def drain_idx(u):
            drain(idx_b[u], isem[u], ids_hbm.at[pl.ds(0, nb)])

        def fire_idx(c, u):
            pltpu.async_copy(
                ids_hbm.at[pl.ds(r_base + c * nb, nb)], idx_b[u], isem[u]
            )

        def fire(c, u):
            drain_idx(u)
            for j in range(nb):
                pltpu.async_copy(
                    v_hbm.at[idx_b[u].at[j]],
                    val_b[u].at[j],
                    gsem[u],
                )

        def process(c, u, head=False):
            drain(val_b[u], gsem[u], out_hbm.at[pl.ds(0, nb)])
            pltpu.async_copy(
                val_b[u], out_hbm.at[pl.ds(r_base + c * nb, nb)], osem[u]
            )
            fire_idx(jnp.minimum(c + depth, nchunk - 1), u)
            cp = c + pf
            up = (u + pf) % depth
            if not head:
                drain(val_b[up], osem[up], out_hbm.at[pl.ds(0, nb)])
            fire(jnp.minimum(cp, nchunk - 1), up)

        for u in range(depth):
            fire_idx(u, u)
        for u in range(pf):
            fire(u, u)
        for u in range(depth):
            process(u, u, head=u < pf)

        def group(g, carry):
            for u in range(depth):
                process(g * depth + u, u)
            return carry

        lax.fori_loop(1, nchunk // depth, group, 0)
        for u in (0, 1):
            drain(val_b[u], gsem[u], out_hbm.at[pl.ds(0, nb)])
        for u in (2, 3):
            drain_idx(u)
            drain(val_b[u], osem[u], out_hbm.at[pl.ds(0, nb)])

    return sc_k(v1d, ids2d)


def kernel(input_ids, s, emb_table, W1, b1, W2, b2):
    B, _, L = input_ids.shape
    vocab = emb_table.shape[0]
    bl = B * L
    assert bl % (128 * _NW) == 0
    vocab_pad = ((vocab + 16383) // 16384) * 16384

    v2d = _tc_vocab_scalar(
        emb_table, W1, b1.reshape(1, 8), W2[:8], b2.reshape(1, 1), vocab_pad
    )
    v1d = v2d.reshape(vocab_pad)
    Lp = ((L + 127) // 128) * 128
    ids2 = input_ids.reshape(B, L)
    ids_pad = jnp.concatenate([ids2, ids2[:, : Lp - L]], axis=1)
    ids2d = ids_pad.reshape(B * Lp // 128, 128)

    g2d = _sc_gather(v1d, ids2d)
    g_pad = g2d.reshape(B, Lp)
    c_2d = W2[8].reshape(1, 1)
    return _tc_axpy_epilogue(g_pad, s, c_2d)
